# baseline (speedup 1.0000x reference)
import functools

import jax
import jax.numpy as jnp
from jax import lax
from jax.experimental import pallas as pl
from jax.experimental.pallas import tpu as pltpu
from jax.experimental.pallas import tpu_sc as plsc

_H = 512
_W = 512
_E = 1024
_NN = 4225
_P = _H * _W
_NC = 2
_NS = 16
_NW = _NC * _NS
_L = 16
_PPT = _P // _NW
_CH = 512
_NCHUNK = _PPT // _CH
_QS = 128
_SUB = 256
_NSUB = _CH // _SUB
_BINS = 1152
_NP = _NN + 15
_GR = _P * 36 // 128
_G = _P * 36 // 16
_RPT = _H // _NW


def _tr_body(qnat, qpix, pin_a, pin_b, pout_v, sem_a, sem_b):
  cid = lax.axis_index("c")
  sid = lax.axis_index("s")
  wid = cid * _NS + sid
  iota = lax.iota(jnp.int32, _L)
  pins = ((pin_a, sem_a), (pin_b, sem_b))

  def _fire(s, pin, sem):
    y0 = wid * _RPT + s * 2
    return [pltpu.async_copy(qnat.at[c, pl.ds(y0, 2)], pin.at[c], sem)
            for c in range(36)]

  descs = _fire(0, pin_a, sem_a)
  for s in range(_RPT // 2):
    pin, _ = pins[s % 2]
    nxt = _fire(s + 1, *pins[(s + 1) % 2]) if s < _RPT // 2 - 1 else []
    for dsc in descs:
      dsc.wait()

    @pl.loop(0, 1024 // _L)
    def _shuffle(v):
      l0 = v * _L
      r = l0 >> 9
      x = l0 & 511
      w = (l0 + iota) * 36
      for c in range(36):
        val = pin[c, r, pl.ds(x, _L)]
        plsc.store_scatter(pout_v, [w + c], val)

    y0 = wid * _RPT + s * 2
    pltpu.sync_copy(pout_v, qpix.at[pl.ds(y0 * _W * 36, 36 * 1024)])
    descs = nxt


def _tec_body(tid_h, xi_h, eta_h, pix_h, elems_h, nu_h, nv_h, q_h,
              out_s, out_c,
              elems_v, nu_v, nv_v,
              tid_v0, xi_v0, eta_v0, pix_v0, oc_v0, qgidx_v0, qrows_v0,
              tid_v1, xi_v1, eta_v1, pix_v1, oc_v1, qgidx_v1, qrows_v1,
              sums_v, cnts_v, rowidx_v,
              shared_s, shared_c, dsem0, dsem1):
  cid = lax.axis_index("c")
  sid = lax.axis_index("s")
  wid = cid * _NS + sid

  iota = lax.iota(jnp.int32, _L)
  zf = jnp.zeros((_L,), jnp.float32)
  bufs = ((tid_v0, xi_v0, eta_v0, pix_v0, oc_v0, qgidx_v0, qrows_v0, dsem0),
          (tid_v1, xi_v1, eta_v1, pix_v1, oc_v1, qgidx_v1, qrows_v1, dsem1))

  pltpu.sync_copy(elems_h, elems_v)
  pltpu.sync_copy(nu_h, nu_v)
  pltpu.sync_copy(nv_h, nv_v)

  @pl.loop(0, _BINS)
  def _zero(r):
    sums_v[r] = zf
    cnts_v[r] = zf

  for k in range(_BINS // _QS):
    for o in range(0, _QS, _L):
      rowidx_v[k, pl.ds(o, _L)] = iota + (k * _QS + o)

  @pl.when(sid == 0)
  def _zero_shared():
    pltpu.sync_copy(sums_v, shared_s)
    pltpu.sync_copy(cnts_v, shared_c)

  base = wid * _PPT

  def _load_phase1_fire(c, buf):
    tid_v, xi_v, eta_v, pix_v, oc_v, qgidx_v, qrows_v, dsem = buf
    off = base + c * _CH
    pltpu.sync_copy(tid_h.at[pl.ds(off, _CH)], tid_v)
    pltpu.sync_copy(xi_h.at[pl.ds(off, _CH)], xi_v)
    pltpu.sync_copy(eta_h.at[pl.ds(off, _CH)], eta_v)
    pltpu.sync_copy(pix_h.at[pl.ds(off, _CH)], pix_v)

    @pl.loop(0, _CH // _L)
    def _phase1(v):
      s16 = pl.ds(v * _L, _L)
      t = tid_v[s16]
      xiv = xi_v[s16]
      etav = eta_v[s16]
      ta = jnp.abs(t)
      e8 = jnp.clip(ta - 1, 0, _E - 1) * 8
      xm = 1.0 - xiv
      xp = 1.0 + xiv
      em = 1.0 - etav
      ep = 1.0 + etav
      xi2 = xiv * xiv
      eta2 = etav * etav
      ns = (
          -0.25 * xm * em * (1.0 + xiv + etav),
          -0.25 * xp * em * (1.0 - xiv + etav),
          -0.25 * xp * ep * (1.0 - xiv - etav),
          -0.25 * xm * ep * (1.0 + xiv - etav),
          0.5 * (1.0 - xi2) * em,
          0.5 * xp * (1.0 - eta2),
          0.5 * (1.0 - xi2) * ep,
          0.5 * xm * (1.0 - eta2),
      )
      u = zf
      w = zf
      for k in range(8):
        ck = plsc.load_gather(elems_v, [e8 + k])
        u = u + ns[k] * plsc.load_gather(nu_v, [ck])
        w = w + ns[k] * plsc.load_gather(nv_v, [ck])
      p = off + v * _L + iota
      gx = jnp.clip(p & (_W - 1), 1, _W - 3).astype(jnp.float32)
      gy = jnp.clip(p >> 9, 1, _H - 3).astype(jnp.float32)
      xs = gx + u
      ys = gy + w
      xt = xs.astype(jnp.int32)
      yt = ys.astype(jnp.int32)
      xf = jnp.where(xs < xt.astype(jnp.float32), xt - 1, xt)
      yf = jnp.where(ys < yt.astype(jnp.float32), yt - 1, yt)
      xf = jnp.clip(xf, 0, _W - 1)
      yf = jnp.clip(yf, 0, _H - 1)
      xi_v[s16] = xs - xf.astype(jnp.float32)
      eta_v[s16] = ys - yf.astype(jnp.float32)
      w0 = (yf * _W + xf) * 36
      g0 = w0 >> 4
      oc_v[s16] = w0 & 15
      p3 = (v * _L + iota) * 3
      plsc.store_scatter(qgidx_v, [p3], g0)
      plsc.store_scatter(qgidx_v, [p3 + 1], g0 + 1)
      plsc.store_scatter(qgidx_v, [p3 + 2], g0 + 2)

    for k in range(3 * _CH // _QS):
      src = qgidx_v.at[pl.ds(k * _QS, _QS)]
      dst = qrows_v.at[pl.ds(k * _QS, _QS)]
      pltpu.async_copy(q_h.at[src], dst, dsem)

  def _drain_phase2(buf):
    tid_v, xi_v, eta_v, pix_v, oc_v, qgidx_v, qrows_v, dsem = buf
    for k in range(3 * _CH // _QS):
      src = qgidx_v.at[pl.ds(k * _QS, _QS)]
      dst = qrows_v.at[pl.ds(k * _QS, _QS)]
      pltpu.make_async_copy(q_h.at[src], dst, dsem).wait()

    @pl.loop(0, _CH // _L)
    def _phase2(v):
      s16 = pl.ds(v * _L, _L)
      xd = xi_v[s16]
      yd = eta_v[s16]
      t = tid_v[s16]
      pv = pix_v[s16]
      oc = oc_v[s16]
      va = jnp.where(t > 0, 1.0, 0.0).astype(jnp.float32)
      wb = (v * _L + iota) * 48 + oc
      y2 = yd * yd
      y3 = y2 * yd
      y4 = y3 * yd
      y5 = y4 * yd
      x2 = xd * xd
      x3 = x2 * xd
      x4 = x3 * xd
      x5 = x4 * xd
      ypows = (None, yd, y2, y3, y4, y5)
      xpows = (None, xd, x2, x3, x4, x5)
      acc = zf
      for j in range(6):
        tj = zf
        for i in range(6):
          w = wb + (i * 6 + j)
          col = plsc.load_gather(qrows_v, [w >> 4, w & 15])
          if i == 0:
            tj = tj + col
          else:
            tj = tj + ypows[i] * col
        if j == 0:
          acc = acc + tj
        else:
          acc = acc + xpows[j] * tj
      r = pv - acc
      r2 = r * r * va
      tc = jnp.clip(jnp.abs(t), 0, _E)
      plsc.addupdate_scatter(sums_v, [tc, iota], r2)
      plsc.addupdate_scatter(cnts_v, [tc, iota], va)

  _load_phase1_fire(0, bufs[0])

  @pl.loop(0, _NCHUNK // 2)
  def _pair(h):
    _load_phase1_fire(2 * h + 1, bufs[1])
    _drain_phase2(bufs[0])

    @pl.when(h < _NCHUNK // 2 - 1)
    def _prefetch():
      _load_phase1_fire(2 * h + 2, bufs[0])

    _drain_phase2(bufs[1])

  plsc.subcore_barrier()
  for k in range(_BINS // _QS):
    pltpu.sync_copy(sums_v.at[pl.ds(k * _QS, _QS)],
                    shared_s.at[rowidx_v.at[k]], add=True)
    pltpu.sync_copy(cnts_v.at[pl.ds(k * _QS, _QS)],
                    shared_c.at[rowidx_v.at[k]], add=True)
  plsc.subcore_barrier()

  @pl.when(sid == 0)
  def _finish():
    pltpu.sync_copy(shared_s, sums_v)
    pltpu.sync_copy(shared_c, cnts_v)

    @pl.loop(0, _BINS // _L)
    def _reduce(g):
      rows = g * _L + iota
      ss = zf
      cc = zf
      for l in range(_L):
        lane = jnp.full((_L,), l, jnp.int32)
        ss = ss + plsc.load_gather(sums_v, [rows, lane])
        cc = cc + plsc.load_gather(cnts_v, [rows, lane])
      qrows_v0[g] = ss
      qrows_v1[g] = cc

    pltpu.sync_copy(qrows_v0.at[pl.ds(0, _BINS // _L)], out_s.at[cid])
    pltpu.sync_copy(qrows_v1.at[pl.ds(0, _BINS // _L)], out_c.at[cid])


@functools.cache
def _get_tr_call():
  return pl.kernel(
      _tr_body,
      out_type=jax.ShapeDtypeStruct((_P * 36,), jnp.float32),
      mesh=plsc.VectorSubcoreMesh(core_axis_name="c", subcore_axis_name="s"),
      compiler_params=pltpu.CompilerParams(needs_layout_passes=False,
                                           use_tc_tiling_on_sc=False),
      scratch_types=[
          pltpu.VMEM((36, 2, _W), jnp.float32),
          pltpu.VMEM((36, 2, _W), jnp.float32),
          pltpu.VMEM((36 * 1024,), jnp.float32),
          pltpu.SemaphoreType.DMA,
          pltpu.SemaphoreType.DMA,
      ],
  )


@functools.cache
def _get_sc_call():
  return pl.kernel(
      _tec_body,
      out_type=(jax.ShapeDtypeStruct((_NC, _BINS // _L, _L), jnp.float32),
                jax.ShapeDtypeStruct((_NC, _BINS // _L, _L), jnp.float32)),
      mesh=plsc.VectorSubcoreMesh(core_axis_name="c", subcore_axis_name="s"),
      compiler_params=pltpu.CompilerParams(needs_layout_passes=False,
                                           use_tc_tiling_on_sc=False),
      scratch_types=[
          pltpu.VMEM((_E * 8,), jnp.int32),
          pltpu.VMEM((_NP,), jnp.float32),
          pltpu.VMEM((_NP,), jnp.float32),
          pltpu.VMEM((_CH,), jnp.int32),
          pltpu.VMEM((_CH,), jnp.float32),
          pltpu.VMEM((_CH,), jnp.float32),
          pltpu.VMEM((_CH,), jnp.float32),
          pltpu.VMEM((_CH,), jnp.int32),
          pltpu.VMEM((3 * _CH,), jnp.int32),
          pltpu.VMEM((3 * _CH, _L), jnp.float32),
          pltpu.VMEM((_CH,), jnp.int32),
          pltpu.VMEM((_CH,), jnp.float32),
          pltpu.VMEM((_CH,), jnp.float32),
          pltpu.VMEM((_CH,), jnp.float32),
          pltpu.VMEM((_CH,), jnp.int32),
          pltpu.VMEM((3 * _CH,), jnp.int32),
          pltpu.VMEM((3 * _CH, _L), jnp.float32),
          pltpu.VMEM((_BINS, _L), jnp.float32),
          pltpu.VMEM((_BINS, _L), jnp.float32),
          pltpu.VMEM((_BINS // _QS, _QS), jnp.int32),
          pltpu.VMEM_SHARED((_BINS, _L), jnp.float32),
          pltpu.VMEM_SHARED((_BINS, _L), jnp.float32),
          pltpu.SemaphoreType.DMA,
          pltpu.SemaphoreType.DMA,
      ],
  )


def kernel(nodes_u, nodes_v, elements, threaddiagram, plot_validpoints,
           plot_global_coords, plot_local_coords, refImg, QKBQKT_def):
  del plot_global_coords
  tid = threaddiagram.reshape(-1)
  val = plot_validpoints.reshape(-1) & (tid > 0)
  tidp = jnp.where(val, tid, -tid)
  xi = plot_local_coords[..., 0].reshape(-1)
  eta = plot_local_coords[..., 1].reshape(-1)
  pix = refImg.reshape(-1)
  elems = elements.reshape(-1)
  nu = jnp.pad(nodes_u, (0, _NP - _NN))
  nv = jnp.pad(nodes_v, (0, _NP - _NN))
  qnat = jnp.transpose(QKBQKT_def, (2, 3, 0, 1)).reshape(36, _H, _W)
  qpix = _get_tr_call()(qnat)
  qg = qpix.reshape(_G, _L)
  sums, cnts = _get_sc_call()(tidp, xi, eta, pix, elems, nu, nv, qg)
  s = sums.reshape(_NC, _BINS).sum(axis=0)
  c = cnts.reshape(_NC, _BINS).sum(axis=0)
  per = jnp.where(c > 0, s / jnp.maximum(c, 1.0), 0.0)
  return jnp.sum(per)


# speedup vs baseline: 1.0027x; 1.0027x over previous
import functools

import jax
import jax.numpy as jnp
from jax import lax
from jax.experimental import pallas as pl
from jax.experimental.pallas import tpu as pltpu
from jax.experimental.pallas import tpu_sc as plsc

_H = 512
_W = 512
_E = 1024
_NN = 4225
_P = _H * _W
_NC = 2
_NS = 16
_NW = _NC * _NS
_L = 16
_PPT = _P // _NW
_CH = 512
_NCHUNK = _PPT // _CH
_QS = 128
_SUB = 256
_NSUB = _CH // _SUB
_BINS = 1152
_NP = _NN + 15
_GR = _P * 36 // 128
_G = _P * 36 // 16
_RPT = _H // _NW


def _tr_body(qnat, qpix, pin_a, pin_b, pout_v, sem_a, sem_b):
  cid = lax.axis_index("c")
  sid = lax.axis_index("s")
  wid = cid * _NS + sid
  iota = lax.iota(jnp.int32, _L)
  pins = ((pin_a, sem_a), (pin_b, sem_b))

  def _fire(s, pin, sem):
    y0 = wid * _RPT + s * 2
    return [pltpu.async_copy(qnat.at[c, pl.ds(y0, 2)], pin.at[c], sem)
            for c in range(36)]

  descs = _fire(0, pin_a, sem_a)
  for s in range(_RPT // 2):
    pin, _ = pins[s % 2]
    nxt = _fire(s + 1, *pins[(s + 1) % 2]) if s < _RPT // 2 - 1 else []
    for dsc in descs:
      dsc.wait()

    @pl.loop(0, 1024 // _L)
    def _shuffle(v):
      l0 = v * _L
      r = l0 >> 9
      x = l0 & 511
      w = (l0 + iota) * 36
      for c in range(36):
        val = pin[c, r, pl.ds(x, _L)]
        wc = w + c
        plsc.store_scatter(pout_v, [wc >> 7, wc & 127], val)

    y0 = wid * _RPT + s * 2
    pltpu.sync_copy(pout_v, qpix.at[pl.ds((y0 * _W * 36) // _QS, 288)])
    descs = nxt


def _tec_body(tid_h, xi_h, eta_h, pix_h, elems_h, nu_h, nv_h, q_h,
              out_s, out_c,
              elems_v, nu_v, nv_v,
              tid_v0, xi_v0, eta_v0, pix_v0, oc_v0, qgidx_v0, qrows_v0,
              tid_v1, xi_v1, eta_v1, pix_v1, oc_v1, qgidx_v1, qrows_v1,
              sums_v, cnts_v, rowidx_v,
              shared_s, shared_c, dsem0, dsem1):
  cid = lax.axis_index("c")
  sid = lax.axis_index("s")
  wid = cid * _NS + sid

  iota = lax.iota(jnp.int32, _L)
  zf = jnp.zeros((_L,), jnp.float32)
  bufs = ((tid_v0, xi_v0, eta_v0, pix_v0, oc_v0, qgidx_v0, qrows_v0, dsem0),
          (tid_v1, xi_v1, eta_v1, pix_v1, oc_v1, qgidx_v1, qrows_v1, dsem1))

  pltpu.sync_copy(elems_h, elems_v)
  pltpu.sync_copy(nu_h, nu_v)
  pltpu.sync_copy(nv_h, nv_v)

  @pl.loop(0, _BINS)
  def _zero(r):
    sums_v[r] = zf
    cnts_v[r] = zf

  for k in range(_BINS // _QS):
    for o in range(0, _QS, _L):
      rowidx_v[k, pl.ds(o, _L)] = iota + (k * _QS + o)

  @pl.when(sid == 0)
  def _zero_shared():
    pltpu.sync_copy(sums_v, shared_s)
    pltpu.sync_copy(cnts_v, shared_c)

  base = wid * _PPT

  def _load_phase1_fire(c, buf):
    tid_v, xi_v, eta_v, pix_v, oc_v, qgidx_v, qrows_v, dsem = buf
    off = base + c * _CH
    pltpu.sync_copy(tid_h.at[pl.ds(off, _CH)], tid_v)
    pltpu.sync_copy(xi_h.at[pl.ds(off, _CH)], xi_v)
    pltpu.sync_copy(eta_h.at[pl.ds(off, _CH)], eta_v)
    pltpu.sync_copy(pix_h.at[pl.ds(off, _CH)], pix_v)

    @pl.loop(0, _CH // _L)
    def _phase1(v):
      s16 = pl.ds(v * _L, _L)
      t = tid_v[s16]
      xiv = xi_v[s16]
      etav = eta_v[s16]
      ta = jnp.abs(t)
      e8 = jnp.clip(ta - 1, 0, _E - 1) * 8
      xm = 1.0 - xiv
      xp = 1.0 + xiv
      em = 1.0 - etav
      ep = 1.0 + etav
      xi2 = xiv * xiv
      eta2 = etav * etav
      ns = (
          -0.25 * xm * em * (1.0 + xiv + etav),
          -0.25 * xp * em * (1.0 - xiv + etav),
          -0.25 * xp * ep * (1.0 - xiv - etav),
          -0.25 * xm * ep * (1.0 + xiv - etav),
          0.5 * (1.0 - xi2) * em,
          0.5 * xp * (1.0 - eta2),
          0.5 * (1.0 - xi2) * ep,
          0.5 * xm * (1.0 - eta2),
      )
      u = zf
      w = zf
      for k in range(8):
        ck = plsc.load_gather(elems_v, [e8 + k])
        u = u + ns[k] * plsc.load_gather(nu_v, [ck])
        w = w + ns[k] * plsc.load_gather(nv_v, [ck])
      p = off + v * _L + iota
      gx = jnp.clip(p & (_W - 1), 1, _W - 3).astype(jnp.float32)
      gy = jnp.clip(p >> 9, 1, _H - 3).astype(jnp.float32)
      xs = gx + u
      ys = gy + w
      xt = xs.astype(jnp.int32)
      yt = ys.astype(jnp.int32)
      xf = jnp.where(xs < xt.astype(jnp.float32), xt - 1, xt)
      yf = jnp.where(ys < yt.astype(jnp.float32), yt - 1, yt)
      xf = jnp.clip(xf, 0, _W - 1)
      yf = jnp.clip(yf, 0, _H - 1)
      xi_v[s16] = xs - xf.astype(jnp.float32)
      eta_v[s16] = ys - yf.astype(jnp.float32)
      w0 = (yf * _W + xf) * 36
      g0 = w0 >> 4
      oc_v[s16] = w0 & 15
      p3 = (v * _L + iota) * 3
      plsc.store_scatter(qgidx_v, [p3], g0)
      plsc.store_scatter(qgidx_v, [p3 + 1], g0 + 1)
      plsc.store_scatter(qgidx_v, [p3 + 2], g0 + 2)

    for k in range(3 * _CH // _QS):
      src = qgidx_v.at[pl.ds(k * _QS, _QS)]
      dst = qrows_v.at[pl.ds(k * _QS, _QS)]
      pltpu.async_copy(q_h.at[src], dst, dsem)

  def _drain_phase2(buf):
    tid_v, xi_v, eta_v, pix_v, oc_v, qgidx_v, qrows_v, dsem = buf
    for k in range(3 * _CH // _QS):
      src = qgidx_v.at[pl.ds(k * _QS, _QS)]
      dst = qrows_v.at[pl.ds(k * _QS, _QS)]
      pltpu.make_async_copy(q_h.at[src], dst, dsem).wait()

    @pl.loop(0, _CH // _L)
    def _phase2(v):
      s16 = pl.ds(v * _L, _L)
      xd = xi_v[s16]
      yd = eta_v[s16]
      t = tid_v[s16]
      pv = pix_v[s16]
      oc = oc_v[s16]
      va = jnp.where(t > 0, 1.0, 0.0).astype(jnp.float32)
      b3 = (v * _L + iota) * 3
      y2 = yd * yd
      y3 = y2 * yd
      y4 = y3 * yd
      y5 = y4 * yd
      x2 = xd * xd
      x3 = x2 * xd
      x4 = x3 * xd
      x5 = x4 * xd
      ypows = (None, yd, y2, y3, y4, y5)
      xpows = (None, xd, x2, x3, x4, x5)
      acc = zf
      for j in range(6):
        tj = zf
        for i in range(6):
          t_ = oc + (i * 6 + j)
          col = plsc.load_gather(qrows_v, [b3 + (t_ >> 4), t_ & 15])
          if i == 0:
            tj = tj + col
          else:
            tj = tj + ypows[i] * col
        if j == 0:
          acc = acc + tj
        else:
          acc = acc + xpows[j] * tj
      r = pv - acc
      r2 = r * r * va
      tc = jnp.clip(jnp.abs(t), 0, _E)
      plsc.addupdate_scatter(sums_v, [tc, iota], r2)
      plsc.addupdate_scatter(cnts_v, [tc, iota], va)

  _load_phase1_fire(0, bufs[0])

  @pl.loop(0, _NCHUNK // 2)
  def _pair(h):
    _load_phase1_fire(2 * h + 1, bufs[1])
    _drain_phase2(bufs[0])

    @pl.when(h < _NCHUNK // 2 - 1)
    def _prefetch():
      _load_phase1_fire(2 * h + 2, bufs[0])

    _drain_phase2(bufs[1])

  plsc.subcore_barrier()
  for k in range(_BINS // _QS):
    pltpu.sync_copy(sums_v.at[pl.ds(k * _QS, _QS)],
                    shared_s.at[rowidx_v.at[k]], add=True)
    pltpu.sync_copy(cnts_v.at[pl.ds(k * _QS, _QS)],
                    shared_c.at[rowidx_v.at[k]], add=True)
  plsc.subcore_barrier()

  @pl.when(sid == 0)
  def _finish():
    pltpu.sync_copy(shared_s, sums_v)
    pltpu.sync_copy(shared_c, cnts_v)

    @pl.loop(0, _BINS // _L)
    def _reduce(g):
      rows = g * _L + iota
      ss = zf
      cc = zf
      for l in range(_L):
        lane = jnp.full((_L,), l, jnp.int32)
        ss = ss + plsc.load_gather(sums_v, [rows, lane])
        cc = cc + plsc.load_gather(cnts_v, [rows, lane])
      qrows_v0[g] = ss
      qrows_v1[g] = cc

    pltpu.sync_copy(qrows_v0.at[pl.ds(0, _BINS // _L)], out_s.at[cid])
    pltpu.sync_copy(qrows_v1.at[pl.ds(0, _BINS // _L)], out_c.at[cid])


@functools.cache
def _get_tr_call():
  return pl.kernel(
      _tr_body,
      out_type=jax.ShapeDtypeStruct((_GR, _QS), jnp.float32),
      mesh=plsc.VectorSubcoreMesh(core_axis_name="c", subcore_axis_name="s"),
      compiler_params=pltpu.CompilerParams(needs_layout_passes=False,
                                           use_tc_tiling_on_sc=False),
      scratch_types=[
          pltpu.VMEM((36, 2, _W), jnp.float32),
          pltpu.VMEM((36, 2, _W), jnp.float32),
          pltpu.VMEM((288, _QS), jnp.float32),
          pltpu.SemaphoreType.DMA,
          pltpu.SemaphoreType.DMA,
      ],
  )


@functools.cache
def _get_sc_call():
  return pl.kernel(
      _tec_body,
      out_type=(jax.ShapeDtypeStruct((_NC, _BINS // _L, _L), jnp.float32),
                jax.ShapeDtypeStruct((_NC, _BINS // _L, _L), jnp.float32)),
      mesh=plsc.VectorSubcoreMesh(core_axis_name="c", subcore_axis_name="s"),
      compiler_params=pltpu.CompilerParams(needs_layout_passes=False,
                                           use_tc_tiling_on_sc=False),
      scratch_types=[
          pltpu.VMEM((_E * 8,), jnp.int32),
          pltpu.VMEM((_NP,), jnp.float32),
          pltpu.VMEM((_NP,), jnp.float32),
          pltpu.VMEM((_CH,), jnp.int32),
          pltpu.VMEM((_CH,), jnp.float32),
          pltpu.VMEM((_CH,), jnp.float32),
          pltpu.VMEM((_CH,), jnp.float32),
          pltpu.VMEM((_CH,), jnp.int32),
          pltpu.VMEM((3 * _CH,), jnp.int32),
          pltpu.VMEM((3 * _CH, _L), jnp.float32),
          pltpu.VMEM((_CH,), jnp.int32),
          pltpu.VMEM((_CH,), jnp.float32),
          pltpu.VMEM((_CH,), jnp.float32),
          pltpu.VMEM((_CH,), jnp.float32),
          pltpu.VMEM((_CH,), jnp.int32),
          pltpu.VMEM((3 * _CH,), jnp.int32),
          pltpu.VMEM((3 * _CH, _L), jnp.float32),
          pltpu.VMEM((_BINS, _L), jnp.float32),
          pltpu.VMEM((_BINS, _L), jnp.float32),
          pltpu.VMEM((_BINS // _QS, _QS), jnp.int32),
          pltpu.VMEM_SHARED((_BINS, _L), jnp.float32),
          pltpu.VMEM_SHARED((_BINS, _L), jnp.float32),
          pltpu.SemaphoreType.DMA,
          pltpu.SemaphoreType.DMA,
      ],
  )


def kernel(nodes_u, nodes_v, elements, threaddiagram, plot_validpoints,
           plot_global_coords, plot_local_coords, refImg, QKBQKT_def):
  del plot_global_coords
  tid = threaddiagram.reshape(-1)
  val = plot_validpoints.reshape(-1) & (tid > 0)
  tidp = jnp.where(val, tid, -tid)
  xi = plot_local_coords[..., 0].reshape(-1)
  eta = plot_local_coords[..., 1].reshape(-1)
  pix = refImg.reshape(-1)
  elems = elements.reshape(-1)
  nu = jnp.pad(nodes_u, (0, _NP - _NN))
  nv = jnp.pad(nodes_v, (0, _NP - _NN))
  qnat = jnp.transpose(QKBQKT_def, (2, 3, 0, 1)).reshape(36, _H, _W)
  qpix = _get_tr_call()(qnat)
  qg = qpix.reshape(_G, _L)
  sums, cnts = _get_sc_call()(tidp, xi, eta, pix, elems, nu, nv, qg)
  s = sums.reshape(_NC, _BINS).sum(axis=0)
  c = cnts.reshape(_NC, _BINS).sum(axis=0)
  per = jnp.where(c > 0, s / jnp.maximum(c, 1.0), 0.0)
  return jnp.sum(per)
